# baseline (speedup 1.0000x reference)
import jax
import jax.numpy as jnp
from jax import lax
from jax.experimental import pallas as pl
from jax.experimental.pallas import tpu as pltpu

_B, _S, _QW, _OW, _H, _M, _K = 8, 16, 2048, 2048, 16, 32768, 32
_HW = _QW // _H
_R = _B * _S
_NG = 512
_NSUB = 16
_SUBW = _M // _NSUB
_NBUF = 2
_MAXREF = 256


def _next_up(x):
    u = lax.bitcast_convert_type(x, jnp.int32)
    up = jnp.where(x >= 0.0, u + 1, u - 1)
    return lax.bitcast_convert_type(up, jnp.float32)


def _mem_copy(mem_hbm, land, sems, h, i):
    return pltpu.make_async_copy(
        mem_hbm.at[h, pl.ds(i * _SUBW, _SUBW), :],
        land.at[i % _NBUF],
        sems.at[i % _NBUF])


def _head_kernel(x_ref, wq_ref, bq_ref, wk_ref, bk_ref, mem_hbm, wo_ref,
                 bo_ref, out_ref, s_ref, cm_ref, memb_ref, land, sems):
    h = pl.program_id(0)
    for i in range(_NBUF):
        _mem_copy(mem_hbm, land, sems, h, i).start()

    x = x_ref[...].astype(jnp.bfloat16)
    wq = wq_ref[...].astype(jnp.bfloat16)
    q = jnp.dot(x, wq, preferred_element_type=jnp.float32) + bq_ref[0, 0][None, :]
    qb = q.astype(jnp.bfloat16)
    wkb = wk_ref[0].astype(jnp.bfloat16)
    bk = bk_ref[0, 0][None, :]

    for i in range(_NSUB):
        _mem_copy(mem_hbm, land, sems, h, i).wait()
        mb = land[i % _NBUF].astype(jnp.bfloat16)
        memb_ref[pl.ds(i * _SUBW, _SUBW), :] = mb
        kf = jnp.dot(mb, wkb, preferred_element_type=jnp.float32) + bk
        kb = kf.astype(jnp.bfloat16)
        blk = jax.lax.dot_general(qb, kb, (((1,), (1,)), ((), ())),
                                  preferred_element_type=jnp.float32)
        s_ref[:, i * _SUBW:(i + 1) * _SUBW] = blk
        if i + _NBUF < _NSUB:
            _mem_copy(mem_hbm, land, sems, h, i + _NBUF).start()
        half = _SUBW // 2
        m1 = jnp.maximum(blk[:, :half], blk[:, half:])
        m2 = jnp.minimum(blk[:, :half], blk[:, half:])
        while half > _NG:
            half //= 2
            a1, b1 = m1[:, :half], m1[:, half:]
            a2, b2 = m2[:, :half], m2[:, half:]
            m1 = jnp.maximum(a1, b1)
            m2 = jnp.maximum(jnp.minimum(a1, b1), jnp.maximum(a2, b2))
        if i == 0:
            cm_ref[:, :_NG] = m1
            cm_ref[:, _NG:] = m2
        else:
            g1 = cm_ref[:, :_NG]
            g2 = cm_ref[:, _NG:]
            cm_ref[:, :_NG] = jnp.maximum(g1, m1)
            cm_ref[:, _NG:] = jnp.maximum(jnp.minimum(g1, m1),
                                          jnp.maximum(g2, m2))

    def cm_body(_, carry):
        cmc, _last = carry
        v = jnp.max(cmc, axis=1, keepdims=True)
        cmc = jnp.where(cmc == v, -jnp.inf, cmc)
        return (cmc, v)
    _, c32 = lax.fori_loop(0, _K, cm_body,
                           (cm_ref[...], jnp.zeros((_R, 1), jnp.float32)))

    sv = s_ref[...]
    cand = sv >= c32
    n0 = jnp.sum(jnp.where(cand, 1.0, 0.0), axis=1, keepdims=True)
    mn0 = jnp.min(jnp.where(cand, sv, jnp.inf), axis=1, keepdims=True)
    fix0 = n0 > _K
    t0 = jnp.where(fix0, _next_up(mn0), c32)
    nn0 = jnp.where(fix0, n0 - 1.0, n0)

    def ref_cond(carry):
        _t, n, it = carry
        return jnp.logical_and(jnp.any(n > _K), it < _MAXREF)

    def ref_body(carry):
        t, n, it = carry
        sv = s_ref[...]
        mn = jnp.min(jnp.where(sv >= t, sv, jnp.inf), axis=1, keepdims=True)
        fix = n > _K
        t2 = jnp.where(fix, _next_up(mn), t)
        n2 = jnp.where(fix, n - 1.0, n)
        return (t2, n2, it + jnp.int32(1))

    t, _n, _ = lax.while_loop(ref_cond, ref_body, (t0, nn0, jnp.int32(0)))

    sv = s_ref[...]
    w = jnp.where(sv >= t, jax.nn.sigmoid(sv), 0.0)
    wb = w.astype(jnp.bfloat16)
    out_h = jnp.dot(wb, memb_ref[...], preferred_element_type=jnp.float32)
    ob = out_h.astype(jnp.bfloat16)
    wob = wo_ref[...].astype(jnp.bfloat16)
    part = jnp.dot(ob, wob, preferred_element_type=jnp.float32)

    @pl.when(h == 0)
    def _():
        out_ref[...] = part + bo_ref[...]

    @pl.when(h != 0)
    def _():
        out_ref[...] = out_ref[...] + part


def kernel(tensor, memory, Wq, bq, Wk, bk, Wo, bo):
    x2d = tensor.reshape(_R, _QW)
    bq_r = bq.reshape(_H, 1, _HW)
    bk_r = bk.reshape(_H, 1, _HW)
    bo_r = bo.reshape(1, _OW)
    out = pl.pallas_call(
        _head_kernel,
        grid=(_H,),
        in_specs=[
            pl.BlockSpec((_R, _QW), lambda h: (0, 0)),
            pl.BlockSpec((_QW, _HW), lambda h: (0, h)),
            pl.BlockSpec((1, 1, _HW), lambda h: (h, 0, 0)),
            pl.BlockSpec((1, _HW, _HW), lambda h: (h, 0, 0)),
            pl.BlockSpec((1, 1, _HW), lambda h: (h, 0, 0)),
            pl.BlockSpec(memory_space=pl.ANY),
            pl.BlockSpec((_HW, _OW), lambda h: (h, 0)),
            pl.BlockSpec((1, _OW), lambda h: (0, 0)),
        ],
        out_specs=pl.BlockSpec((_R, _OW), lambda h: (0, 0)),
        out_shape=jax.ShapeDtypeStruct((_R, _OW), jnp.float32),
        scratch_shapes=[
            pltpu.VMEM((_R, _M), jnp.float32),
            pltpu.VMEM((_R, 2 * _NG), jnp.float32),
            pltpu.VMEM((_M, _HW), jnp.bfloat16),
            pltpu.VMEM((_NBUF, _SUBW, _HW), jnp.float32),
            pltpu.SemaphoreType.DMA((_NBUF,)),
        ],
        compiler_params=pltpu.CompilerParams(
            dimension_semantics=("arbitrary",),
        ),
    )(x2d, Wq, bq_r, Wk, bk_r, memory, Wo, bo_r)
    return out.reshape(_B, _S, _OW)


# speedup vs baseline: 1.0228x; 1.0228x over previous
import jax
import jax.numpy as jnp
from jax import lax
from jax.experimental import pallas as pl
from jax.experimental.pallas import tpu as pltpu

_B, _S, _QW, _OW, _H, _M, _K = 8, 16, 2048, 2048, 16, 32768, 32
_HW = _QW // _H
_R = _B * _S
_NG = 256
_NSUB = 16
_SUBW = _M // _NSUB
_NBUF = 2
_MAXREF = 256


def _next_up(x):
    u = lax.bitcast_convert_type(x, jnp.int32)
    up = jnp.where(x >= 0.0, u + 1, u - 1)
    return lax.bitcast_convert_type(up, jnp.float32)


def _mem_copy(mem_hbm, land, sems, h, i):
    return pltpu.make_async_copy(
        mem_hbm.at[h, pl.ds(i * _SUBW, _SUBW), :],
        land.at[i % _NBUF],
        sems.at[i % _NBUF])


def _head_kernel(x_ref, wq_ref, bq_ref, wk_ref, bk_ref, mem_hbm, wo_ref,
                 bo_ref, out_ref, s_ref, cm_ref, memb_ref, land, sems):
    h = pl.program_id(0)
    for i in range(_NBUF):
        _mem_copy(mem_hbm, land, sems, h, i).start()

    x = x_ref[...].astype(jnp.bfloat16)
    wq = wq_ref[...].astype(jnp.bfloat16)
    q = jnp.dot(x, wq, preferred_element_type=jnp.float32) + bq_ref[0, 0][None, :]
    qb = q.astype(jnp.bfloat16)
    wkb = wk_ref[0].astype(jnp.bfloat16)
    bk = bk_ref[0, 0][None, :]

    for i in range(_NSUB):
        _mem_copy(mem_hbm, land, sems, h, i).wait()
        mb = land[i % _NBUF].astype(jnp.bfloat16)
        memb_ref[pl.ds(i * _SUBW, _SUBW), :] = mb
        kf = jnp.dot(mb, wkb, preferred_element_type=jnp.float32) + bk
        kb = kf.astype(jnp.bfloat16)
        blk = jax.lax.dot_general(qb, kb, (((1,), (1,)), ((), ())),
                                  preferred_element_type=jnp.float32)
        s_ref[:, i * _SUBW:(i + 1) * _SUBW] = blk
        if i + _NBUF < _NSUB:
            _mem_copy(mem_hbm, land, sems, h, i + _NBUF).start()
        half = _SUBW // 2
        m1 = jnp.maximum(blk[:, :half], blk[:, half:])
        m2 = jnp.minimum(blk[:, :half], blk[:, half:])
        while half > _NG:
            half //= 2
            a1, b1 = m1[:, :half], m1[:, half:]
            a2, b2 = m2[:, :half], m2[:, half:]
            m1 = jnp.maximum(a1, b1)
            m2 = jnp.maximum(jnp.minimum(a1, b1), jnp.maximum(a2, b2))
        if i == 0:
            cm_ref[:, :_NG] = m1
            cm_ref[:, _NG:] = m2
        else:
            g1 = cm_ref[:, :_NG]
            g2 = cm_ref[:, _NG:]
            cm_ref[:, :_NG] = jnp.maximum(g1, m1)
            cm_ref[:, _NG:] = jnp.maximum(jnp.minimum(g1, m1),
                                          jnp.maximum(g2, m2))

    def cm_body(_, carry):
        cmc, _last = carry
        v = jnp.max(cmc, axis=1, keepdims=True)
        cmc = jnp.where(cmc == v, -jnp.inf, cmc)
        return (cmc, v)
    _, c32 = lax.fori_loop(0, _K, cm_body,
                           (cm_ref[...], jnp.zeros((_R, 1), jnp.float32)))

    sv = s_ref[...]
    cand = sv >= c32
    n0 = jnp.sum(jnp.where(cand, 1.0, 0.0), axis=1, keepdims=True)
    mn0 = jnp.min(jnp.where(cand, sv, jnp.inf), axis=1, keepdims=True)
    fix0 = n0 > _K
    t0 = jnp.where(fix0, _next_up(mn0), c32)
    nn0 = jnp.where(fix0, n0 - 1.0, n0)

    def ref_cond(carry):
        _t, n, it = carry
        return jnp.logical_and(jnp.any(n > _K), it < _MAXREF)

    def ref_body(carry):
        t, n, it = carry
        sv = s_ref[...]
        mn = jnp.min(jnp.where(sv >= t, sv, jnp.inf), axis=1, keepdims=True)
        fix = n > _K
        t2 = jnp.where(fix, _next_up(mn), t)
        n2 = jnp.where(fix, n - 1.0, n)
        return (t2, n2, it + jnp.int32(1))

    t, _n, _ = lax.while_loop(ref_cond, ref_body, (t0, nn0, jnp.int32(0)))

    sv = s_ref[...]
    w = jnp.where(sv >= t, jax.nn.sigmoid(sv), 0.0)
    wb = w.astype(jnp.bfloat16)
    out_h = jnp.dot(wb, memb_ref[...], preferred_element_type=jnp.float32)
    ob = out_h.astype(jnp.bfloat16)
    wob = wo_ref[...].astype(jnp.bfloat16)
    part = jnp.dot(ob, wob, preferred_element_type=jnp.float32)

    @pl.when(h == 0)
    def _():
        out_ref[...] = part + bo_ref[...]

    @pl.when(h != 0)
    def _():
        out_ref[...] = out_ref[...] + part


def kernel(tensor, memory, Wq, bq, Wk, bk, Wo, bo):
    x2d = tensor.reshape(_R, _QW)
    bq_r = bq.reshape(_H, 1, _HW)
    bk_r = bk.reshape(_H, 1, _HW)
    bo_r = bo.reshape(1, _OW)
    out = pl.pallas_call(
        _head_kernel,
        grid=(_H,),
        in_specs=[
            pl.BlockSpec((_R, _QW), lambda h: (0, 0)),
            pl.BlockSpec((_QW, _HW), lambda h: (0, h)),
            pl.BlockSpec((1, 1, _HW), lambda h: (h, 0, 0)),
            pl.BlockSpec((1, _HW, _HW), lambda h: (h, 0, 0)),
            pl.BlockSpec((1, 1, _HW), lambda h: (h, 0, 0)),
            pl.BlockSpec(memory_space=pl.ANY),
            pl.BlockSpec((_HW, _OW), lambda h: (h, 0)),
            pl.BlockSpec((1, _OW), lambda h: (0, 0)),
        ],
        out_specs=pl.BlockSpec((_R, _OW), lambda h: (0, 0)),
        out_shape=jax.ShapeDtypeStruct((_R, _OW), jnp.float32),
        scratch_shapes=[
            pltpu.VMEM((_R, _M), jnp.float32),
            pltpu.VMEM((_R, 2 * _NG), jnp.float32),
            pltpu.VMEM((_M, _HW), jnp.bfloat16),
            pltpu.VMEM((_NBUF, _SUBW, _HW), jnp.float32),
            pltpu.SemaphoreType.DMA((_NBUF,)),
        ],
        compiler_params=pltpu.CompilerParams(
            dimension_semantics=("arbitrary",),
        ),
    )(x2d, Wq, bq_r, Wk, bk_r, memory, Wo, bo_r)
    return out.reshape(_B, _S, _OW)
